# Initial kernel scaffold; baseline (speedup 1.0000x reference)
import jax
import jax.numpy as jnp
from jax.experimental import pallas as pl


def kernel(position_matrix, channel_matrix, space_points_num, outpoint_num, kernel_coeffs, bias):
    raise NotImplementedError("write your pallas kernel here")


# speedup vs baseline: 2.0155x; 2.0155x over previous
import functools

import jax
import jax.numpy as jnp
from jax.experimental import pallas as pl

IN_CH = 16
OUT_CH = 32
NLM = 27
K = 16
SPN = 2048
OUTN = 512
QT = 128


def _body(posT_ref, cq_ref, packed_ref, kc2_ref, tile_ref, red_ref, bias_ref,
          centers_ref, out_ref, resnet_ref):
    posT = posT_ref[0]
    cq = cq_ref[0]
    packed = packed_ref[0]
    f32 = jnp.float32

    e0 = cq[:, 0:1] - posT[0:1, :]
    e1 = cq[:, 1:2] - posT[1:2, :]
    e2 = cq[:, 2:3] - posT[2:3, :]
    d = (e0 * e0 + e1 * e1) + e2 * e2

    iota = jax.lax.broadcasted_iota(jnp.int32, (QT, SPN), 1)
    inf = jnp.float32(jnp.inf)
    big = jnp.int32(SPN + 1)

    gathered = []
    csum = jnp.zeros((QT, 3), dtype=f32)
    for _ in range(K):
        m = jnp.min(d, axis=1, keepdims=True)
        idx = jnp.min(jnp.where(d == m, iota, big), axis=1, keepdims=True)
        onehot_b = iota == idx
        d = jnp.where(onehot_b, inf, d)
        onehot = onehot_b.astype(f32)
        g = jnp.dot(onehot, packed, preferred_element_type=f32, precision=jax.lax.Precision.HIGHEST)
        gathered.append(g)
        csum = csum + g[:, 0:3]

    centers = csum * jnp.float32(1.0 / K)
    centers_ref[0] = centers

    kc2 = kc2_ref[...]
    tile = tile_ref[...]
    red = red_ref[...]

    eps = jnp.float32(1e-8)
    acc = jnp.zeros((QT, OUT_CH * IN_CH), dtype=f32)
    fsum = jnp.zeros((QT, 1), dtype=f32)
    for k in range(K):
        g = gathered[k]
        local = g[:, 0:3] - centers
        feats = g[:, 3:3 + IN_CH]
        s = (local[:, 0:1] * local[:, 0:1] + local[:, 1:2] * local[:, 1:2]) \
            + local[:, 2:3] * local[:, 2:3] + eps
        r = jnp.sqrt(s)
        ct1 = jnp.clip(local[:, 2:3] / r, -0.999999, 0.999999)
        ct2 = jnp.float32(2.0) * ct1 * ct1 - jnp.float32(1.0)
        xp = local[:, 0:1] + eps
        yy = local[:, 1:2]
        rho = jnp.maximum(jnp.sqrt(xp * xp + yy * yy), jnp.float32(1e-30))
        cp1 = xp / rho
        cp2 = jnp.float32(2.0) * cp1 * cp1 - jnp.float32(1.0)
        one = jnp.ones((QT, 1), dtype=f32)
        rp = (one, r, s)
        ct = (one, ct1, ct2)
        cp = (one, cp1, cp2)
        cols = []
        for n in range(3):
            for l in range(3):
                for mth in range(3):
                    cols.append(rp[n] * ct[l] * cp[mth])
        basis = jnp.concatenate(cols, axis=1)
        w = jnp.dot(basis, kc2, preferred_element_type=f32, precision=jax.lax.Precision.HIGHEST)
        ft = jnp.dot(feats, tile, preferred_element_type=f32, precision=jax.lax.Precision.HIGHEST)
        acc = acc + w * ft
        fsum = fsum + jnp.sum(feats, axis=1, keepdims=True)

    out = jnp.dot(acc, red, preferred_element_type=f32, precision=jax.lax.Precision.HIGHEST) + bias_ref[...]
    out_ref[0] = out
    resnet_ref[0] = jnp.broadcast_to(fsum, (QT, OUT_CH))


@functools.partial(jax.jit, static_argnums=(2, 3))
def _run(position_matrix, channel_matrix, spn, outn, kernel_coeffs, bias):
    total = position_matrix.shape[0]
    s = total // spn
    pos3 = position_matrix.reshape(s, spn, 3)
    stride = spn // outn
    cq = pos3[:, ::stride, :]
    posT = jnp.transpose(pos3, (0, 2, 1))
    packed = jnp.concatenate(
        [pos3, channel_matrix.reshape(s, spn, IN_CH)], axis=-1)
    kc2 = jnp.transpose(kernel_coeffs, (2, 0, 1)).reshape(NLM, OUT_CH * IN_CH)
    eye = jnp.eye(IN_CH, dtype=jnp.float32)
    tile = jnp.tile(eye, (1, OUT_CH))
    red = jnp.repeat(jnp.eye(OUT_CH, dtype=jnp.float32), IN_CH, axis=0)
    bias2 = bias.reshape(1, OUT_CH)

    grid = (s, outn // QT)
    out_shapes = (
        jax.ShapeDtypeStruct((s, outn, 3), jnp.float32),
        jax.ShapeDtypeStruct((s, outn, OUT_CH), jnp.float32),
        jax.ShapeDtypeStruct((s, outn, OUT_CH), jnp.float32),
    )
    centers, out, resnet = pl.pallas_call(
        _body,
        grid=grid,
        in_specs=[
            pl.BlockSpec((1, 3, spn), lambda i, j: (i, 0, 0)),
            pl.BlockSpec((1, QT, 3), lambda i, j: (i, j, 0)),
            pl.BlockSpec((1, spn, 3 + IN_CH), lambda i, j: (i, 0, 0)),
            pl.BlockSpec((NLM, OUT_CH * IN_CH), lambda i, j: (0, 0)),
            pl.BlockSpec((IN_CH, OUT_CH * IN_CH), lambda i, j: (0, 0)),
            pl.BlockSpec((OUT_CH * IN_CH, OUT_CH), lambda i, j: (0, 0)),
            pl.BlockSpec((1, OUT_CH), lambda i, j: (0, 0)),
        ],
        out_specs=(
            pl.BlockSpec((1, QT, 3), lambda i, j: (i, j, 0)),
            pl.BlockSpec((1, QT, OUT_CH), lambda i, j: (i, j, 0)),
            pl.BlockSpec((1, QT, OUT_CH), lambda i, j: (i, j, 0)),
        ),
        out_shape=out_shapes,
    )(posT, cq, packed, kc2, tile, red, bias2)

    p = s * outn
    return (centers.reshape(p, 3), out.reshape(p, OUT_CH),
            resnet.reshape(p, OUT_CH))


def kernel(position_matrix, channel_matrix, space_points_num, outpoint_num,
           kernel_coeffs, bias):
    return _run(position_matrix, channel_matrix, SPN, OUTN,
                kernel_coeffs, bias)


# speedup vs baseline: 3.2169x; 1.5960x over previous
import functools

import jax
import jax.numpy as jnp
from jax.experimental import pallas as pl

IN_CH = 16
OUT_CH = 32
NLM = 27
K = 16
SPN = 2048
OUTN = 512
QT = 256


def _body(posT_ref, cq_ref, ph_ref, pm_ref, plo_ref, kc2_ref, tile_ref,
          red_ref, bias_ref, centers_ref, out_ref, resnet_ref):
    posT = posT_ref[0]
    cq = cq_ref[0]
    ph = ph_ref[0]
    pm = pm_ref[0]
    plo = plo_ref[0]
    f32 = jnp.float32

    e0 = cq[:, 0:1] - posT[0:1, :]
    e1 = cq[:, 1:2] - posT[1:2, :]
    e2 = cq[:, 2:3] - posT[2:3, :]
    d = (e0 * e0 + e1 * e1) + e2 * e2

    iota = jax.lax.broadcasted_iota(jnp.int32, (QT, SPN), 1)
    inf = jnp.float32(jnp.inf)
    big = jnp.int32(SPN + 1)

    gathered = []
    csum = jnp.zeros((QT, 3), dtype=f32)
    for _ in range(K):
        m = jnp.min(d, axis=1, keepdims=True)
        idx = jnp.min(jnp.where(d == m, iota, big), axis=1, keepdims=True)
        onehot_b = iota == idx
        d = jnp.where(onehot_b, inf, d)
        onehot = onehot_b.astype(jnp.bfloat16)
        gh = jnp.dot(onehot, ph, preferred_element_type=f32)
        gm = jnp.dot(onehot, pm, preferred_element_type=f32)
        gl = jnp.dot(onehot, plo, preferred_element_type=f32)
        g = (gh + gm) + gl
        gathered.append(g)
        csum = csum + g[:, 0:3]

    centers = csum * jnp.float32(1.0 / K)
    centers_ref[0] = centers

    kc2 = kc2_ref[...]
    tile = tile_ref[...]

    eps = jnp.float32(1e-8)
    acc = jnp.zeros((QT, OUT_CH * IN_CH), dtype=f32)
    fsum = jnp.zeros((QT, 1), dtype=f32)
    for k in range(K):
        g = gathered[k]
        local = g[:, 0:3] - centers
        feats = g[:, 3:3 + IN_CH]
        s = (local[:, 0:1] * local[:, 0:1] + local[:, 1:2] * local[:, 1:2]) \
            + local[:, 2:3] * local[:, 2:3] + eps
        r = jnp.sqrt(s)
        ct1 = jnp.clip(local[:, 2:3] / r, -0.999999, 0.999999)
        ct2 = jnp.float32(2.0) * ct1 * ct1 - jnp.float32(1.0)
        xp = local[:, 0:1] + eps
        yy = local[:, 1:2]
        rho = jnp.maximum(jnp.sqrt(xp * xp + yy * yy), jnp.float32(1e-30))
        cp1 = xp / rho
        cp2 = jnp.float32(2.0) * cp1 * cp1 - jnp.float32(1.0)
        cpv = jnp.concatenate([jnp.ones((QT, 1), dtype=f32), cp1, cp2],
                              axis=1)
        ctcp = jnp.concatenate([cpv, ct1 * cpv, ct2 * cpv], axis=1)
        basis = jnp.concatenate([ctcp, r * ctcp, s * ctcp], axis=1)
        w = jnp.dot(basis, kc2, preferred_element_type=f32, precision=jax.lax.Precision.HIGHEST)
        ft = jnp.dot(feats, tile, preferred_element_type=f32, precision=jax.lax.Precision.HIGHEST)
        acc = acc + w * ft
        fsum = fsum + jnp.sum(feats, axis=1, keepdims=True)

    red_b = red_ref[...]
    acc_h = acc.astype(jnp.bfloat16)
    acc_l = (acc - acc_h.astype(f32)).astype(jnp.bfloat16)
    out = (jnp.dot(acc_h, red_b, preferred_element_type=f32)
           + jnp.dot(acc_l, red_b, preferred_element_type=f32)) + bias_ref[...]
    out_ref[0] = out
    resnet_ref[0] = jnp.broadcast_to(fsum, (QT, OUT_CH))


@functools.partial(jax.jit, static_argnums=(2, 3))
def _run(position_matrix, channel_matrix, spn, outn, kernel_coeffs, bias):
    total = position_matrix.shape[0]
    s = total // spn
    pos3 = position_matrix.reshape(s, spn, 3)
    stride = spn // outn
    cq = pos3[:, ::stride, :]
    posT = jnp.transpose(pos3, (0, 2, 1))
    packed = jnp.concatenate(
        [pos3, channel_matrix.reshape(s, spn, IN_CH)], axis=-1)
    ph = packed.astype(jnp.bfloat16)
    rem1 = packed - ph.astype(jnp.float32)
    pm = rem1.astype(jnp.bfloat16)
    plo = (rem1 - pm.astype(jnp.float32)).astype(jnp.bfloat16)
    kc2 = jnp.transpose(kernel_coeffs, (2, 0, 1)).reshape(NLM, OUT_CH * IN_CH)
    eye = jnp.eye(IN_CH, dtype=jnp.float32)
    tile = jnp.tile(eye, (1, OUT_CH))
    red = jnp.repeat(jnp.eye(OUT_CH, dtype=jnp.bfloat16), IN_CH, axis=0)
    bias2 = bias.reshape(1, OUT_CH)

    grid = (s, outn // QT)
    out_shapes = (
        jax.ShapeDtypeStruct((s, outn, 3), jnp.float32),
        jax.ShapeDtypeStruct((s, outn, OUT_CH), jnp.float32),
        jax.ShapeDtypeStruct((s, outn, OUT_CH), jnp.float32),
    )
    centers, out, resnet = pl.pallas_call(
        _body,
        grid=grid,
        in_specs=[
            pl.BlockSpec((1, 3, spn), lambda i, j: (i, 0, 0)),
            pl.BlockSpec((1, QT, 3), lambda i, j: (i, j, 0)),
            pl.BlockSpec((1, spn, 3 + IN_CH), lambda i, j: (i, 0, 0)),
            pl.BlockSpec((1, spn, 3 + IN_CH), lambda i, j: (i, 0, 0)),
            pl.BlockSpec((1, spn, 3 + IN_CH), lambda i, j: (i, 0, 0)),
            pl.BlockSpec((NLM, OUT_CH * IN_CH), lambda i, j: (0, 0)),
            pl.BlockSpec((IN_CH, OUT_CH * IN_CH), lambda i, j: (0, 0)),
            pl.BlockSpec((OUT_CH * IN_CH, OUT_CH), lambda i, j: (0, 0)),
            pl.BlockSpec((1, OUT_CH), lambda i, j: (0, 0)),
        ],
        out_specs=(
            pl.BlockSpec((1, QT, 3), lambda i, j: (i, j, 0)),
            pl.BlockSpec((1, QT, OUT_CH), lambda i, j: (i, j, 0)),
            pl.BlockSpec((1, QT, OUT_CH), lambda i, j: (i, j, 0)),
        ),
        out_shape=out_shapes,
    )(posT, cq, ph, pm, plo, kc2, tile, red, bias2)

    p = s * outn
    return (centers.reshape(p, 3), out.reshape(p, OUT_CH),
            resnet.reshape(p, OUT_CH))


def kernel(position_matrix, channel_matrix, space_points_num, outpoint_num,
           kernel_coeffs, bias):
    return _run(position_matrix, channel_matrix, SPN, OUTN,
                kernel_coeffs, bias)


# speedup vs baseline: 3.5675x; 1.1090x over previous
import functools

import jax
import jax.numpy as jnp
from jax.experimental import pallas as pl

IN_CH = 16
OUT_CH = 32
NLM = 27
K = 16
SPN = 2048
OUTN = 512
QT = 256


def _body(posT_ref, cq_ref, ph_ref, pm_ref, plo_ref, kc2_ref, tile_ref,
          red_ref, bias_ref, centers_ref, out_ref, resnet_ref):
    posT = posT_ref[0]
    cq = cq_ref[0]
    ph = ph_ref[0]
    pm = pm_ref[0]
    plo = plo_ref[0]
    f32 = jnp.float32

    e0 = cq[:, 0:1] - posT[0:1, :]
    e1 = cq[:, 1:2] - posT[1:2, :]
    e2 = cq[:, 2:3] - posT[2:3, :]
    d = (e0 * e0 + e1 * e1) + e2 * e2

    iota = jax.lax.broadcasted_iota(jnp.int32, (QT, SPN), 1)
    inf = jnp.float32(jnp.inf)
    big = jnp.int32(SPN + 1)

    gathered = []
    csum = jnp.zeros((QT, 3), dtype=f32)
    for _ in range(K):
        m = jnp.min(d, axis=1, keepdims=True)
        idx = jnp.min(jnp.where(d == m, iota, big), axis=1, keepdims=True)
        onehot_b = iota == idx
        d = jnp.where(onehot_b, inf, d)
        onehot = onehot_b.astype(f32)
        gh = jnp.dot(onehot, ph, preferred_element_type=f32)
        gm = jnp.dot(onehot, pm, preferred_element_type=f32)
        gl = jnp.dot(onehot, plo, preferred_element_type=f32)
        g = (gh + gm) + gl
        gathered.append(g)
        csum = csum + g[:, 0:3]

    centers = csum * jnp.float32(1.0 / K)
    centers_ref[0] = centers

    kc2 = kc2_ref[...]
    tile = tile_ref[...]

    eps = jnp.float32(1e-8)
    acc = jnp.zeros((QT, OUT_CH * IN_CH), dtype=f32)
    fsum = jnp.zeros((QT, 1), dtype=f32)
    for k in range(K):
        g = gathered[k]
        local = g[:, 0:3] - centers
        feats = g[:, 3:3 + IN_CH]
        s = (local[:, 0:1] * local[:, 0:1] + local[:, 1:2] * local[:, 1:2]) \
            + local[:, 2:3] * local[:, 2:3] + eps
        r = jnp.sqrt(s)
        ct1 = jnp.clip(local[:, 2:3] / r, -0.999999, 0.999999)
        ct2 = jnp.float32(2.0) * ct1 * ct1 - jnp.float32(1.0)
        xp = local[:, 0:1] + eps
        yy = local[:, 1:2]
        rho = jnp.maximum(jnp.sqrt(xp * xp + yy * yy), jnp.float32(1e-30))
        cp1 = xp / rho
        cp2 = jnp.float32(2.0) * cp1 * cp1 - jnp.float32(1.0)
        cpv = jnp.concatenate([jnp.ones((QT, 1), dtype=f32), cp1, cp2],
                              axis=1)
        ctcp = jnp.concatenate([cpv, ct1 * cpv, ct2 * cpv], axis=1)
        basis = jnp.concatenate([ctcp, r * ctcp, s * ctcp], axis=1)
        w = jnp.dot(basis, kc2, preferred_element_type=f32, precision=jax.lax.Precision.HIGHEST)
        ft = jnp.dot(feats, tile, preferred_element_type=f32, precision=jax.lax.Precision.HIGHEST)
        acc = acc + w * ft
        fsum = fsum + jnp.sum(feats, axis=1, keepdims=True)

    red_b = red_ref[...]
    mask = jnp.int32(-65536)
    acc_h = jax.lax.bitcast_convert_type(
        jax.lax.bitcast_convert_type(acc, jnp.int32) & mask, f32)
    acc_l = acc - acc_h
    out = (jnp.dot(acc_h, red_b, preferred_element_type=f32)
           + jnp.dot(acc_l, red_b, preferred_element_type=f32)) + bias_ref[...]
    out_ref[0] = out
    resnet_ref[0] = jnp.broadcast_to(fsum, (QT, OUT_CH))


@functools.partial(jax.jit, static_argnums=(2, 3))
def _run(position_matrix, channel_matrix, spn, outn, kernel_coeffs, bias):
    total = position_matrix.shape[0]
    s = total // spn
    pos3 = position_matrix.reshape(s, spn, 3)
    stride = spn // outn
    cq = pos3[:, ::stride, :]
    posT = jnp.transpose(pos3, (0, 2, 1))
    packed = jnp.concatenate(
        [pos3, channel_matrix.reshape(s, spn, IN_CH)], axis=-1)
    mask = jnp.int32(-65536)
    ph = jax.lax.bitcast_convert_type(
        jax.lax.bitcast_convert_type(packed, jnp.int32) & mask, jnp.float32)
    rem1 = packed - ph
    pm = jax.lax.bitcast_convert_type(
        jax.lax.bitcast_convert_type(rem1, jnp.int32) & mask, jnp.float32)
    plo = rem1 - pm
    kc2 = jnp.transpose(kernel_coeffs, (2, 0, 1)).reshape(NLM, OUT_CH * IN_CH)
    eye = jnp.eye(IN_CH, dtype=jnp.float32)
    tile = jnp.tile(eye, (1, OUT_CH))
    red = jnp.repeat(jnp.eye(OUT_CH, dtype=jnp.float32), IN_CH, axis=0)
    bias2 = bias.reshape(1, OUT_CH)

    grid = (s, outn // QT)
    out_shapes = (
        jax.ShapeDtypeStruct((s, outn, 3), jnp.float32),
        jax.ShapeDtypeStruct((s, outn, OUT_CH), jnp.float32),
        jax.ShapeDtypeStruct((s, outn, OUT_CH), jnp.float32),
    )
    centers, out, resnet = pl.pallas_call(
        _body,
        grid=grid,
        in_specs=[
            pl.BlockSpec((1, 3, spn), lambda i, j: (i, 0, 0)),
            pl.BlockSpec((1, QT, 3), lambda i, j: (i, j, 0)),
            pl.BlockSpec((1, spn, 3 + IN_CH), lambda i, j: (i, 0, 0)),
            pl.BlockSpec((1, spn, 3 + IN_CH), lambda i, j: (i, 0, 0)),
            pl.BlockSpec((1, spn, 3 + IN_CH), lambda i, j: (i, 0, 0)),
            pl.BlockSpec((NLM, OUT_CH * IN_CH), lambda i, j: (0, 0)),
            pl.BlockSpec((IN_CH, OUT_CH * IN_CH), lambda i, j: (0, 0)),
            pl.BlockSpec((OUT_CH * IN_CH, OUT_CH), lambda i, j: (0, 0)),
            pl.BlockSpec((1, OUT_CH), lambda i, j: (0, 0)),
        ],
        out_specs=(
            pl.BlockSpec((1, QT, 3), lambda i, j: (i, j, 0)),
            pl.BlockSpec((1, QT, OUT_CH), lambda i, j: (i, j, 0)),
            pl.BlockSpec((1, QT, OUT_CH), lambda i, j: (i, j, 0)),
        ),
        out_shape=out_shapes,
    )(posT, cq, ph, pm, plo, kc2, tile, red, bias2)

    p = s * outn
    return (centers.reshape(p, 3), out.reshape(p, OUT_CH),
            resnet.reshape(p, OUT_CH))


def kernel(position_matrix, channel_matrix, space_points_num, outpoint_num,
           kernel_coeffs, bias):
    return _run(position_matrix, channel_matrix, SPN, OUTN,
                kernel_coeffs, bias)


# speedup vs baseline: 4.7504x; 1.3316x over previous
import functools

import jax
import jax.numpy as jnp
from jax.experimental import pallas as pl

IN_CH = 16
OUT_CH = 32
NLM = 27
K = 16
SPN = 2048
OUTN = 512
QT = 256
MASK = -65536


def _hi(x):
    return jax.lax.bitcast_convert_type(
        jax.lax.bitcast_convert_type(x, jnp.int32) & jnp.int32(MASK),
        jnp.float32)


def _body(posT_ref, cq_ref, packed3_ref, kc2h_ref, kc2l_ref, tile_ref,
          red_ref, bias_ref, centers_ref, out_ref, resnet_ref):
    posT = posT_ref[0]
    cq = cq_ref[0]
    packed3 = packed3_ref[0]
    f32 = jnp.float32

    e0 = cq[:, 0:1] - posT[0:1, :]
    e1 = cq[:, 1:2] - posT[1:2, :]
    e2 = cq[:, 2:3] - posT[2:3, :]
    d = (e0 * e0 + e1 * e1) + e2 * e2

    iota = jax.lax.broadcasted_iota(jnp.int32, (QT, SPN), 1)
    inf = jnp.float32(jnp.inf)
    big = jnp.int32(SPN + 1)

    gathered = []
    csum = jnp.zeros((QT, 3), dtype=f32)
    for _ in range(K):
        m = jnp.min(d, axis=1, keepdims=True)
        idx = jnp.min(jnp.where(d == m, iota, big), axis=1, keepdims=True)
        onehot_b = iota == idx
        d = jnp.where(onehot_b, inf, d)
        onehot = onehot_b.astype(f32)
        g3 = jnp.dot(onehot, packed3, preferred_element_type=f32)
        g = (g3[:, 0:32] + g3[:, 32:64]) + g3[:, 64:96]
        gathered.append(g)
        csum = csum + g[:, 0:3]

    centers = csum * jnp.float32(1.0 / K)
    centers_ref[0] = centers

    eps = jnp.float32(1e-8)
    gx = jnp.concatenate([g[:, 0:1] for g in gathered], axis=1)
    gy = jnp.concatenate([g[:, 1:2] for g in gathered], axis=1)
    gz = jnp.concatenate([g[:, 2:3] for g in gathered], axis=1)
    lx = gx - centers[:, 0:1]
    ly = gy - centers[:, 1:2]
    lz = gz - centers[:, 2:3]
    S = (lx * lx + ly * ly) + lz * lz + eps
    R = jnp.sqrt(S)
    CT1 = jnp.clip(lz / R, -0.999999, 0.999999)
    CT2 = jnp.float32(2.0) * CT1 * CT1 - jnp.float32(1.0)
    xp = lx + eps
    rho = jnp.maximum(jnp.sqrt(xp * xp + ly * ly), jnp.float32(1e-30))
    CP1 = xp / rho
    CP2 = jnp.float32(2.0) * CP1 * CP1 - jnp.float32(1.0)

    kc2h = kc2h_ref[...]
    kc2l = kc2l_ref[...]
    tile = tile_ref[...]
    one = jnp.ones((QT, 1), dtype=f32)

    acc = jnp.zeros((QT, OUT_CH * IN_CH), dtype=f32)
    fsum = jnp.zeros((QT, 1), dtype=f32)
    for k in range(K):
        cpv = jnp.concatenate([one, CP1[:, k:k + 1], CP2[:, k:k + 1]],
                              axis=1)
        ctcp = jnp.concatenate(
            [cpv, CT1[:, k:k + 1] * cpv, CT2[:, k:k + 1] * cpv], axis=1)
        basis = jnp.concatenate(
            [ctcp, R[:, k:k + 1] * ctcp, S[:, k:k + 1] * ctcp], axis=1)
        bh = _hi(basis)
        bl = basis - bh
        w = (jnp.dot(bh, kc2h, preferred_element_type=f32)
             + jnp.dot(bh, kc2l, preferred_element_type=f32)
             + jnp.dot(bl, kc2h, preferred_element_type=f32))
        feats = gathered[k][:, 3:3 + IN_CH]
        fh = _hi(feats)
        fl = feats - fh
        ft = (jnp.dot(fh, tile, preferred_element_type=f32)
              + jnp.dot(fl, tile, preferred_element_type=f32))
        acc = acc + w * ft
        fsum = fsum + jnp.sum(feats, axis=1, keepdims=True)

    red = red_ref[...]
    acc_h = _hi(acc)
    acc_l = acc - acc_h
    out = (jnp.dot(acc_h, red, preferred_element_type=f32)
           + jnp.dot(acc_l, red, preferred_element_type=f32)) + bias_ref[...]
    out_ref[0] = out
    resnet_ref[0] = jnp.broadcast_to(fsum, (QT, OUT_CH))


@functools.partial(jax.jit, static_argnums=(2, 3))
def _run(position_matrix, channel_matrix, spn, outn, kernel_coeffs, bias):
    total = position_matrix.shape[0]
    s = total // spn
    f32 = jnp.float32
    pos3 = position_matrix.reshape(s, spn, 3)
    stride = spn // outn
    cq = pos3[:, ::stride, :]
    posT = jnp.transpose(pos3, (0, 2, 1))
    packed = jnp.concatenate(
        [pos3, channel_matrix.reshape(s, spn, IN_CH)], axis=-1)
    mask = jnp.int32(MASK)
    ph = jax.lax.bitcast_convert_type(
        jax.lax.bitcast_convert_type(packed, jnp.int32) & mask, f32)
    rem1 = packed - ph
    pm = jax.lax.bitcast_convert_type(
        jax.lax.bitcast_convert_type(rem1, jnp.int32) & mask, f32)
    plo = rem1 - pm
    zpad = jnp.zeros((s, spn, 32 - (3 + IN_CH)), dtype=f32)
    packed3 = jnp.concatenate([ph, zpad, pm, zpad, plo, zpad], axis=-1)

    kc2 = jnp.transpose(kernel_coeffs, (2, 0, 1)).reshape(NLM, OUT_CH * IN_CH)
    kc2h = jax.lax.bitcast_convert_type(
        jax.lax.bitcast_convert_type(kc2, jnp.int32) & mask, f32)
    kc2l = kc2 - kc2h
    eye = jnp.eye(IN_CH, dtype=f32)
    tile = jnp.tile(eye, (1, OUT_CH))
    red = jnp.repeat(jnp.eye(OUT_CH, dtype=f32), IN_CH, axis=0)
    bias2 = bias.reshape(1, OUT_CH)

    grid = (s, outn // QT)
    out_shapes = (
        jax.ShapeDtypeStruct((s, outn, 3), jnp.float32),
        jax.ShapeDtypeStruct((s, outn, OUT_CH), jnp.float32),
        jax.ShapeDtypeStruct((s, outn, OUT_CH), jnp.float32),
    )
    centers, out, resnet = pl.pallas_call(
        _body,
        grid=grid,
        in_specs=[
            pl.BlockSpec((1, 3, spn), lambda i, j: (i, 0, 0)),
            pl.BlockSpec((1, QT, 3), lambda i, j: (i, j, 0)),
            pl.BlockSpec((1, spn, 96), lambda i, j: (i, 0, 0)),
            pl.BlockSpec((NLM, OUT_CH * IN_CH), lambda i, j: (0, 0)),
            pl.BlockSpec((NLM, OUT_CH * IN_CH), lambda i, j: (0, 0)),
            pl.BlockSpec((IN_CH, OUT_CH * IN_CH), lambda i, j: (0, 0)),
            pl.BlockSpec((OUT_CH * IN_CH, OUT_CH), lambda i, j: (0, 0)),
            pl.BlockSpec((1, OUT_CH), lambda i, j: (0, 0)),
        ],
        out_specs=(
            pl.BlockSpec((1, QT, 3), lambda i, j: (i, j, 0)),
            pl.BlockSpec((1, QT, OUT_CH), lambda i, j: (i, j, 0)),
            pl.BlockSpec((1, QT, OUT_CH), lambda i, j: (i, j, 0)),
        ),
        out_shape=out_shapes,
    )(posT, cq, packed3, kc2h, kc2l, tile, red, bias2)

    p = s * outn
    return (centers.reshape(p, 3), out.reshape(p, OUT_CH),
            resnet.reshape(p, OUT_CH))


def kernel(position_matrix, channel_matrix, space_points_num, outpoint_num,
           kernel_coeffs, bias):
    return _run(position_matrix, channel_matrix, SPN, OUTN,
                kernel_coeffs, bias)
